# baseline (speedup 1.0000x reference)
import jax
import jax.numpy as jnp
from jax.experimental import pallas as pl

_BB = 64


def _body(x_ref, m_ref, o_ref):
    x = x_ref[...]
    m = m_ref[...].astype(x.dtype)
    s = jnp.sum(x * m[:, :, None], axis=1)
    c = jnp.sum(m, axis=1, keepdims=True)
    o_ref[...] = s / c


def kernel(inputs, mask):
    B, L, D = inputs.shape
    grid = (B // _BB,)
    return pl.pallas_call(
        _body,
        grid=grid,
        in_specs=[
            pl.BlockSpec((_BB, L, D), lambda i: (i, 0, 0)),
            pl.BlockSpec((_BB, L), lambda i: (i, 0)),
        ],
        out_specs=pl.BlockSpec((_BB, D), lambda i: (i, 0)),
        out_shape=jax.ShapeDtypeStruct((B, D), inputs.dtype),
    )(inputs, mask.view(jnp.uint8))


# speedup vs baseline: 1.1881x; 1.1881x over previous
import jax
import jax.numpy as jnp
from jax import lax
from jax.experimental import pallas as pl
from jax.experimental.pallas import tpu as pltpu

_CH = 32
_NBUF = 8


def _body(x_hbm, m_ref, o_ref, *scratch):
    bufs = scratch[:_NBUF]
    sems = scratch[_NBUF:]
    B = o_ref.shape[0]
    nchunks = B // _CH

    def dma(c, b):
        return pltpu.make_async_copy(
            x_hbm.at[pl.ds(c * _CH, _CH)], bufs[b], sems[b]
        )

    for b in range(_NBUF):
        dma(b, b).start()

    def outer(k, _):
        c0 = k * _NBUF
        for b in range(_NBUF):
            c = c0 + b
            dma(c, b).wait()
            x = bufs[b][...]
            m = m_ref[pl.ds(c * _CH, _CH), :].astype(x.dtype)
            s = jnp.sum(x * m[:, :, None], axis=1)
            cnt = jnp.sum(m, axis=1, keepdims=True)
            o_ref[pl.ds(c * _CH, _CH), :] = s / cnt

            @pl.when(c + _NBUF < nchunks)
            def _():
                dma(c + _NBUF, b).start()

        return 0

    lax.fori_loop(0, nchunks // _NBUF, outer, 0)


def kernel(inputs, mask):
    B, L, D = inputs.shape
    return pl.pallas_call(
        _body,
        in_specs=[
            pl.BlockSpec(memory_space=pl.ANY),
            pl.BlockSpec((B, L), lambda: (0, 0)),
        ],
        out_specs=pl.BlockSpec((B, D), lambda: (0, 0)),
        out_shape=jax.ShapeDtypeStruct((B, D), inputs.dtype),
        scratch_shapes=(
            [pltpu.VMEM((_CH, L, D), inputs.dtype) for _ in range(_NBUF)]
            + [pltpu.SemaphoreType.DMA for _ in range(_NBUF)]
        ),
    )(inputs, mask.view(jnp.uint8))


# speedup vs baseline: 1.2429x; 1.0461x over previous
import jax
import jax.numpy as jnp
from jax import lax
from jax.experimental import pallas as pl
from jax.experimental.pallas import tpu as pltpu

_CH = 64
_NBUF = 4


def _body(x_hbm, m_ref, o_ref, *scratch):
    bufs = scratch[:_NBUF]
    sems = scratch[_NBUF:]
    B = o_ref.shape[0]
    nchunks = B // _CH

    def dma(c, b):
        return pltpu.make_async_copy(
            x_hbm.at[pl.ds(c * _CH, _CH)], bufs[b], sems[b]
        )

    for b in range(_NBUF):
        dma(b, b).start()

    def outer(k, _):
        c0 = k * _NBUF
        for b in range(_NBUF):
            c = c0 + b
            dma(c, b).wait()
            x = bufs[b][...]
            m = m_ref[pl.ds(c * _CH, _CH), :].astype(x.dtype)
            s = jnp.sum(x * m[:, :, None], axis=1)
            cnt = jnp.sum(m, axis=1, keepdims=True)
            o_ref[pl.ds(c * _CH, _CH), :] = s / cnt

            @pl.when(c + _NBUF < nchunks)
            def _():
                dma(c + _NBUF, b).start()

        return 0

    lax.fori_loop(0, nchunks // _NBUF, outer, 0)


def kernel(inputs, mask):
    B, L, D = inputs.shape
    return pl.pallas_call(
        _body,
        in_specs=[
            pl.BlockSpec(memory_space=pl.ANY),
            pl.BlockSpec((B, L), lambda: (0, 0)),
        ],
        out_specs=pl.BlockSpec((B, D), lambda: (0, 0)),
        out_shape=jax.ShapeDtypeStruct((B, D), inputs.dtype),
        scratch_shapes=(
            [pltpu.VMEM((_CH, L, D), inputs.dtype) for _ in range(_NBUF)]
            + [pltpu.SemaphoreType.DMA for _ in range(_NBUF)]
        ),
    )(inputs, mask.view(jnp.uint8))
